# baseline (speedup 1.0000x reference)
import functools

import jax
import jax.numpy as jnp
from jax import lax
from jax.experimental import pallas as pl
from jax.experimental.pallas import tpu as pltpu
from jax.experimental.pallas import tpu_sc as plsc

RATIO = 0.25
KNN = 32


@functools.lru_cache(maxsize=None)
def _sc_gather_build(n_rows, d, nw, ch):
    rows_pw = n_rows // nw
    n_chunks = rows_pw // ch
    mesh = plsc.VectorSubcoreMesh(core_axis_name="c", subcore_axis_name="s")

    @functools.partial(
        pl.kernel,
        out_type=jax.ShapeDtypeStruct((n_rows, d), jnp.float32),
        mesh=mesh,
        scratch_types=[
            pltpu.VMEM((ch,), jnp.int32),
            pltpu.VMEM((ch, d), jnp.float32),
            pltpu.SemaphoreType.DMA,
        ],
    )
    def k(tab_hbm, idx_hbm, out_hbm, idx_v, rows_v, sem):
        wid = lax.axis_index("s") * 2 + lax.axis_index("c")
        base = wid * rows_pw
        for j in range(n_chunks):
            off = base + j * ch
            pltpu.sync_copy(idx_hbm.at[pl.ds(off, ch)], idx_v)
            pltpu.async_copy(tab_hbm.at[idx_v], rows_v, sem).wait()
            pltpu.sync_copy(rows_v, out_hbm.at[pl.ds(off, ch)])

    return k


def _fps_body(pos_ref, idx_ref):
    B, _, N = pos_ref.shape
    S = idx_ref.shape[1]
    x = pos_ref[:, 0, :]
    y = pos_ref[:, 1, :]
    z = pos_ref[:, 2, :]
    iota_n = jax.lax.broadcasted_iota(jnp.int32, (B, N), 1)
    col_s = jax.lax.broadcasted_iota(jnp.int32, (B, S), 1)

    def body(i, st):
        dist, lx, ly, lz, acc = st
        dx = x - lx
        dy = y - ly
        dz = z - lz
        d = (dx * dx + dy * dy) + dz * dz
        dist = jnp.minimum(dist, d)
        m = jnp.max(dist, axis=1, keepdims=True)
        newidx = jnp.min(jnp.where(dist == m, iota_n, N), axis=1, keepdims=True)
        pm = iota_n == newidx
        lx = jnp.sum(jnp.where(pm, x, 0.0), axis=1, keepdims=True)
        ly = jnp.sum(jnp.where(pm, y, 0.0), axis=1, keepdims=True)
        lz = jnp.sum(jnp.where(pm, z, 0.0), axis=1, keepdims=True)
        acc = jnp.where(col_s == i, newidx, acc)
        return (dist, lx, ly, lz, acc)

    dist0 = jnp.full((B, N), jnp.inf, dtype=jnp.float32)
    acc0 = jnp.zeros((B, S), dtype=jnp.int32)
    st = jax.lax.fori_loop(
        1, S, body, (dist0, x[:, 0:1], y[:, 0:1], z[:, 0:1], acc0))
    idx_ref[...] = st[4]


def _fps_pallas(pose):
    B, N, _ = pose.shape
    S = int(N * RATIO)
    posT = jnp.transpose(pose, (0, 2, 1))
    return pl.pallas_call(
        _fps_body,
        out_shape=jax.ShapeDtypeStruct((B, S), jnp.int32),
    )(posT)


def _knn_body(cent_ref, pos_ref, out_ref):
    _, _, SB = cent_ref.shape
    N = pos_ref.shape[2]
    b = pl.program_id(0)
    cx = cent_ref[0, 0, :][:, None]
    cy = cent_ref[0, 1, :][:, None]
    cz = cent_ref[0, 2, :][:, None]
    x = pos_ref[0, 0, :][None, :]
    y = pos_ref[0, 1, :][None, :]
    z = pos_ref[0, 2, :][None, :]
    dx = cx - x
    dy = cy - y
    dz = cz - z
    d2 = (dx * dx + dy * dy) + dz * dz
    iota_n = jax.lax.broadcasted_iota(jnp.int32, (SB, N), 1)
    col_k = jax.lax.broadcasted_iota(jnp.int32, (SB, KNN), 1)
    out = jnp.zeros((SB, KNN), jnp.int32)
    for k in range(KNN):
        m = jnp.min(d2, axis=1, keepdims=True)
        idx = jnp.min(jnp.where(d2 == m, iota_n, N), axis=1, keepdims=True)
        out = jnp.where(col_k == k, idx, out)
        d2 = jnp.where(iota_n == idx, jnp.inf, d2)
    out_ref[0] = out + b * N


def _knn_pallas(new_pose, pose):
    B, S, _ = new_pose.shape
    N = pose.shape[1]
    SB = 512
    centT = jnp.transpose(new_pose, (0, 2, 1))
    posT = jnp.transpose(pose, (0, 2, 1))
    return pl.pallas_call(
        _knn_body,
        grid=(B, S // SB),
        in_specs=[
            pl.BlockSpec((1, 3, SB), lambda b, s: (b, 0, s)),
            pl.BlockSpec((1, 3, N), lambda b, s: (b, 0, 0)),
        ],
        out_specs=pl.BlockSpec((1, SB, KNN), lambda b, s: (b, s, 0)),
        out_shape=jax.ShapeDtypeStruct((B, S, KNN), jnp.int32),
    )(centT, posT)


def _bf(v):
    return v.astype(jnp.bfloat16).astype(jnp.float32)


def _bdot(a, b):
    return jnp.dot(a.astype(jnp.bfloat16), b.astype(jnp.bfloat16),
                   preferred_element_type=jnp.float32)


def _p1_body(cl_ref, cent_ref, w1_ref, b1_ref, w2_ref, b2_ref,
             wc1a_ref, wc1b_ref, bc1_ref, y1_ref, st_ref):
    BM = cl_ref.shape[0]
    G = cent_ref.shape[0]
    cl = cl_ref[...]
    feat = cl[:, 0:32]
    cent = cent_ref[:, 32:38]
    cb = jnp.reshape(jnp.broadcast_to(cent[:, None, :], (G, KNN, 6)), (BM, 6))
    dx = cl[:, 32:33] - cb[:, 0:1]
    dy = cl[:, 33:34] - cb[:, 1:2]
    dz = cl[:, 34:35] - cb[:, 2:3]
    dist = jnp.sqrt((dx * dx + dy * dy) + dz * dz)
    inv = 1.0 / (dist + 1e-8)
    ux, uy, uz = dx * inv, dy * inv, dz * inv
    ax, ay, az = cb[:, 3:4], cb[:, 4:5], cb[:, 5:6]
    bx, by, bz = cl[:, 35:36], cl[:, 36:37], cl[:, 37:38]

    def angle(px, py, pz, qx, qy, qz):
        cx = py * qz - pz * qy
        cy = pz * qx - px * qz
        cz = px * qy - py * qx
        cn = jnp.sqrt((cx * cx + cy * cy) + cz * cz)
        dt = (px * qx + py * qy) + pz * qz
        return jnp.arctan2(cn, dt)

    zero = dist == 0.0
    f1 = jnp.where(zero, 0.0, angle(ax, ay, az, ux, uy, uz))
    f2 = jnp.where(zero, 0.0, angle(bx, by, bz, ux, uy, uz))
    f3 = angle(ax, ay, az, bx, by, bz)
    w1 = w1_ref[...]
    h = (_bf(f1) * _bf(w1[0:1, :]) + _bf(f2) * _bf(w1[1:2, :])
         + _bf(f3) * _bf(w1[2:3, :]) + _bf(dist) * _bf(w1[3:4, :])
         + _bf(dist) * _bf(w1[4:5, :]) + b1_ref[...])
    h = jnp.maximum(h, 0.0)
    frri = jnp.maximum(_bdot(h, w2_ref[...]) + b2_ref[...], 0.0)
    y1 = _bdot(feat, wc1a_ref[...]) + _bdot(frri, wc1b_ref[...]) + bc1_ref[...]
    y1_ref[...] = y1
    st_ref[0, 0, :] = jnp.sum(y1, axis=0)
    st_ref[0, 1, :] = jnp.sum(y1 * y1, axis=0)


def _p2_body(y1_ref, sc_ref, sh_ref, w_ref, b_ref, y2_ref, st_ref):
    x1 = jnp.maximum(y1_ref[...] * sc_ref[...] + sh_ref[...], 0.0)
    y2 = _bdot(x1, w_ref[...]) + b_ref[...]
    y2_ref[...] = y2
    st_ref[0, 0, :] = jnp.sum(y2, axis=0)
    st_ref[0, 1, :] = jnp.sum(y2 * y2, axis=0)


def _p3_body(y2_ref, sc_ref, sh_ref, o_ref):
    BM = y2_ref.shape[0]
    G = BM // KNN
    x2 = jnp.maximum(y2_ref[...] * sc_ref[...] + sh_ref[...], 0.0)
    o_ref[...] = jnp.max(jnp.reshape(x2, (G, KNN, 128)), axis=1)


def kernel(pointCloudPose, featureVector, PointCloudNormal, SH,
           rri_W1, rri_b1, rri_W2, rri_b2,
           conv_W1, conv_b1, bn_g1, bn_b1,
           conv_W2, conv_b2, bn_g2, bn_b2):
    B, N, _ = pointCloudPose.shape
    S = int(N * RATIO)
    fps_idx = _fps_pallas(pointCloudPose)

    zc = jnp.zeros((B, N, 128 - 47), jnp.float32)
    tab = jnp.concatenate(
        [featureVector, pointCloudPose, PointCloudNormal, SH, zc], -1
    ).reshape(B * N, 128)
    boff = (jnp.arange(B, dtype=jnp.int32) * N)
    fps_flat = (fps_idx + boff[:, None]).reshape(-1)

    cent_rows = _sc_gather_build(B * S, 128, 32, 128)(tab, fps_flat)
    new_pose = cent_rows[:, 32:35].reshape(B, S, 3)
    new_normal = cent_rows[:, 35:38].reshape(B, S, 3)
    new_sh = cent_rows[:, 38:47].reshape(B, S, 9)

    nn_flat = _knn_pallas(new_pose, pointCloudPose).reshape(-1)
    cl_rows = _sc_gather_build(B * S * KNN, 128, 32, 512)(tab, nn_flat)

    M = B * S * KNN
    BM = 4096
    G = M // BM
    row2 = lambda a: a.reshape(1, -1)

    y1, st1 = pl.pallas_call(
        _p1_body,
        grid=(G,),
        in_specs=[
            pl.BlockSpec((BM, 128), lambda i: (i, 0)),
            pl.BlockSpec((BM // KNN, 128), lambda i: (i, 0)),
            pl.BlockSpec((5, 64), lambda i: (0, 0)),
            pl.BlockSpec((1, 64), lambda i: (0, 0)),
            pl.BlockSpec((64, 64), lambda i: (0, 0)),
            pl.BlockSpec((1, 64), lambda i: (0, 0)),
            pl.BlockSpec((32, 64), lambda i: (0, 0)),
            pl.BlockSpec((64, 64), lambda i: (0, 0)),
            pl.BlockSpec((1, 64), lambda i: (0, 0)),
        ],
        out_specs=[
            pl.BlockSpec((BM, 64), lambda i: (i, 0)),
            pl.BlockSpec((1, 2, 64), lambda i: (i, 0, 0)),
        ],
        out_shape=[
            jax.ShapeDtypeStruct((M, 64), jnp.float32),
            jax.ShapeDtypeStruct((G, 2, 64), jnp.float32),
        ],
    )(cl_rows, cent_rows, rri_W1, row2(rri_b1), rri_W2, row2(rri_b2),
      conv_W1[0:32], conv_W1[32:96], row2(conv_b1))

    s1 = jnp.sum(st1, axis=0)
    mean1 = s1[0] / M
    var1 = s1[1] / M - mean1 * mean1
    sc1 = bn_g1 / jnp.sqrt(var1 + 1e-5)
    sh1 = bn_b1 - mean1 * sc1

    def bn_mm_pass(y, sc, sh, W, b, cout):
        return pl.pallas_call(
            _p2_body,
            grid=(G,),
            in_specs=[
                pl.BlockSpec((BM, y.shape[1]), lambda i: (i, 0)),
                pl.BlockSpec((1, y.shape[1]), lambda i: (0, 0)),
                pl.BlockSpec((1, y.shape[1]), lambda i: (0, 0)),
                pl.BlockSpec((y.shape[1], cout), lambda i: (0, 0)),
                pl.BlockSpec((1, cout), lambda i: (0, 0)),
            ],
            out_specs=[
                pl.BlockSpec((BM, cout), lambda i: (i, 0)),
                pl.BlockSpec((1, 2, cout), lambda i: (i, 0, 0)),
            ],
            out_shape=[
                jax.ShapeDtypeStruct((M, cout), jnp.float32),
                jax.ShapeDtypeStruct((G, 2, cout), jnp.float32),
            ],
        )(y, row2(sc), row2(sh), W, row2(b))

    y2, st2 = bn_mm_pass(y1, sc1, sh1, conv_W2, conv_b2, 128)
    s2 = jnp.sum(st2, axis=0)
    mean2 = s2[0] / M
    var2 = s2[1] / M - mean2 * mean2
    sc2 = bn_g2 / jnp.sqrt(var2 + 1e-5)
    sh2 = bn_b2 - mean2 * sc2

    new_feat = pl.pallas_call(
        _p3_body,
        grid=(G,),
        in_specs=[
            pl.BlockSpec((BM, 128), lambda i: (i, 0)),
            pl.BlockSpec((1, 128), lambda i: (0, 0)),
            pl.BlockSpec((1, 128), lambda i: (0, 0)),
        ],
        out_specs=pl.BlockSpec((BM // KNN, 128), lambda i: (i, 0)),
        out_shape=jax.ShapeDtypeStruct((B * S, 128), jnp.float32),
    )(y2, row2(sc2), row2(sh2)).reshape(B, S, 128)
    return (new_pose, new_feat, new_normal, new_sh)


# speedup vs baseline: 1.0004x; 1.0004x over previous
import functools

import jax
import jax.numpy as jnp
from jax import lax
from jax.experimental import pallas as pl
from jax.experimental.pallas import tpu as pltpu
from jax.experimental.pallas import tpu_sc as plsc

RATIO = 0.25
KNN = 32


@functools.lru_cache(maxsize=None)
def _sc_gather_build(n_rows, d, nw, ch):
    rows_pw = n_rows // nw
    n_chunks = rows_pw // ch
    mesh = plsc.VectorSubcoreMesh(core_axis_name="c", subcore_axis_name="s")

    @functools.partial(
        pl.kernel,
        out_type=jax.ShapeDtypeStruct((n_rows, d), jnp.float32),
        mesh=mesh,
        scratch_types=[
            pltpu.VMEM((ch,), jnp.int32),
            pltpu.VMEM((ch, d), jnp.float32),
            pltpu.SemaphoreType.DMA,
        ],
    )
    def k(tab_hbm, idx_hbm, out_hbm, idx_v, rows_v, sem):
        wid = lax.axis_index("s") * 2 + lax.axis_index("c")
        base = wid * rows_pw
        for j in range(n_chunks):
            off = base + j * ch
            pltpu.sync_copy(idx_hbm.at[pl.ds(off, ch)], idx_v)
            pltpu.async_copy(tab_hbm.at[idx_v], rows_v, sem).wait()
            pltpu.sync_copy(rows_v, out_hbm.at[pl.ds(off, ch)])

    return k


def _fps_body(pos_ref, idx_ref):
    B, _, N = pos_ref.shape
    S = idx_ref.shape[1]
    x = pos_ref[:, 0, :]
    y = pos_ref[:, 1, :]
    z = pos_ref[:, 2, :]
    iota_n = jax.lax.broadcasted_iota(jnp.int32, (B, N), 1)
    col_s = jax.lax.broadcasted_iota(jnp.int32, (B, S), 1)

    def body(i, st):
        dist, lx, ly, lz, acc = st
        dx = x - lx
        dy = y - ly
        dz = z - lz
        d = (dx * dx + dy * dy) + dz * dz
        dist = jnp.minimum(dist, d)
        m = jnp.max(dist, axis=1, keepdims=True)
        newidx = jnp.min(jnp.where(dist == m, iota_n, N), axis=1, keepdims=True)
        pm = iota_n == newidx
        lx = jnp.sum(jnp.where(pm, x, 0.0), axis=1, keepdims=True)
        ly = jnp.sum(jnp.where(pm, y, 0.0), axis=1, keepdims=True)
        lz = jnp.sum(jnp.where(pm, z, 0.0), axis=1, keepdims=True)
        acc = jnp.where(col_s == i, newidx, acc)
        return (dist, lx, ly, lz, acc)

    dist0 = jnp.full((B, N), jnp.inf, dtype=jnp.float32)
    acc0 = jnp.zeros((B, S), dtype=jnp.int32)
    st = jax.lax.fori_loop(
        1, S, body, (dist0, x[:, 0:1], y[:, 0:1], z[:, 0:1], acc0))
    idx_ref[...] = st[4]


def _fps_pallas(pose):
    B, N, _ = pose.shape
    S = int(N * RATIO)
    posT = jnp.transpose(pose, (0, 2, 1))
    return pl.pallas_call(
        _fps_body,
        out_shape=jax.ShapeDtypeStruct((B, S), jnp.int32),
    )(posT)


def _knn_body(cent_ref, pos_ref, out_ref):
    _, _, SB = cent_ref.shape
    N = pos_ref.shape[2]
    b = pl.program_id(0)
    cx = cent_ref[0, 0, :][:, None]
    cy = cent_ref[0, 1, :][:, None]
    cz = cent_ref[0, 2, :][:, None]
    x = pos_ref[0, 0, :][None, :]
    y = pos_ref[0, 1, :][None, :]
    z = pos_ref[0, 2, :][None, :]
    dx = cx - x
    dy = cy - y
    dz = cz - z
    d2 = (dx * dx + dy * dy) + dz * dz
    iota_n = jax.lax.broadcasted_iota(jnp.int32, (SB, N), 1)
    col_k = jax.lax.broadcasted_iota(jnp.int32, (SB, KNN), 1)
    out = jnp.zeros((SB, KNN), jnp.int32)
    for k in range(KNN):
        m = jnp.min(d2, axis=1, keepdims=True)
        idx = jnp.min(jnp.where(d2 == m, iota_n, N), axis=1, keepdims=True)
        out = jnp.where(col_k == k, idx, out)
        d2 = jnp.where(iota_n == idx, jnp.inf, d2)
    out_ref[0] = out + b * N


def _knn_pallas(new_pose, pose):
    B, S, _ = new_pose.shape
    N = pose.shape[1]
    SB = 512
    centT = jnp.transpose(new_pose, (0, 2, 1))
    posT = jnp.transpose(pose, (0, 2, 1))
    return pl.pallas_call(
        _knn_body,
        grid=(B, S // SB),
        in_specs=[
            pl.BlockSpec((1, 3, SB), lambda b, s: (b, 0, s)),
            pl.BlockSpec((1, 3, N), lambda b, s: (b, 0, 0)),
        ],
        out_specs=pl.BlockSpec((1, SB, KNN), lambda b, s: (b, s, 0)),
        out_shape=jax.ShapeDtypeStruct((B, S, KNN), jnp.int32),
        compiler_params=pltpu.CompilerParams(
            dimension_semantics=("parallel", "parallel")),
    )(centT, posT)


def _bf(v):
    return v.astype(jnp.bfloat16).astype(jnp.float32)


def _bdot(a, b):
    return jnp.dot(a.astype(jnp.bfloat16), b.astype(jnp.bfloat16),
                   preferred_element_type=jnp.float32)


def _p1_body(cl_ref, cent_ref, w1_ref, b1_ref, w2_ref, b2_ref,
             wc1a_ref, wc1b_ref, bc1_ref, y1_ref, st_ref):
    BM = cl_ref.shape[0]
    G = cent_ref.shape[0]
    cl = cl_ref[...]
    feat = cl[:, 0:32]
    cent = cent_ref[:, 32:38]
    cb = jnp.reshape(jnp.broadcast_to(cent[:, None, :], (G, KNN, 6)), (BM, 6))
    dx = cl[:, 32:33] - cb[:, 0:1]
    dy = cl[:, 33:34] - cb[:, 1:2]
    dz = cl[:, 34:35] - cb[:, 2:3]
    dist = jnp.sqrt((dx * dx + dy * dy) + dz * dz)
    inv = 1.0 / (dist + 1e-8)
    ux, uy, uz = dx * inv, dy * inv, dz * inv
    ax, ay, az = cb[:, 3:4], cb[:, 4:5], cb[:, 5:6]
    bx, by, bz = cl[:, 35:36], cl[:, 36:37], cl[:, 37:38]

    def angle(px, py, pz, qx, qy, qz):
        cx = py * qz - pz * qy
        cy = pz * qx - px * qz
        cz = px * qy - py * qx
        cn = jnp.sqrt((cx * cx + cy * cy) + cz * cz)
        dt = (px * qx + py * qy) + pz * qz
        return jnp.arctan2(cn, dt)

    zero = dist == 0.0
    f1 = jnp.where(zero, 0.0, angle(ax, ay, az, ux, uy, uz))
    f2 = jnp.where(zero, 0.0, angle(bx, by, bz, ux, uy, uz))
    f3 = angle(ax, ay, az, bx, by, bz)
    w1 = w1_ref[...]
    h = (_bf(f1) * _bf(w1[0:1, :]) + _bf(f2) * _bf(w1[1:2, :])
         + _bf(f3) * _bf(w1[2:3, :]) + _bf(dist) * _bf(w1[3:4, :])
         + _bf(dist) * _bf(w1[4:5, :]) + b1_ref[...])
    h = jnp.maximum(h, 0.0)
    frri = jnp.maximum(_bdot(h, w2_ref[...]) + b2_ref[...], 0.0)
    y1 = _bdot(feat, wc1a_ref[...]) + _bdot(frri, wc1b_ref[...]) + bc1_ref[...]
    y1_ref[...] = y1
    st_ref[0, 0, :] = jnp.sum(y1, axis=0)
    st_ref[0, 1, :] = jnp.sum(y1 * y1, axis=0)


def _p2_body(y1_ref, sc_ref, sh_ref, w_ref, b_ref, y2_ref, st_ref):
    x1 = jnp.maximum(y1_ref[...] * sc_ref[...] + sh_ref[...], 0.0)
    y2 = _bdot(x1, w_ref[...]) + b_ref[...]
    y2_ref[...] = y2
    st_ref[0, 0, :] = jnp.sum(y2, axis=0)
    st_ref[0, 1, :] = jnp.sum(y2 * y2, axis=0)


def _p3_body(y2_ref, sc_ref, sh_ref, o_ref):
    BM = y2_ref.shape[0]
    G = BM // KNN
    x2 = jnp.maximum(y2_ref[...] * sc_ref[...] + sh_ref[...], 0.0)
    o_ref[...] = jnp.max(jnp.reshape(x2, (G, KNN, 128)), axis=1)


def kernel(pointCloudPose, featureVector, PointCloudNormal, SH,
           rri_W1, rri_b1, rri_W2, rri_b2,
           conv_W1, conv_b1, bn_g1, bn_b1,
           conv_W2, conv_b2, bn_g2, bn_b2):
    B, N, _ = pointCloudPose.shape
    S = int(N * RATIO)
    fps_idx = _fps_pallas(pointCloudPose)

    zc = jnp.zeros((B, N, 128 - 47), jnp.float32)
    tab = jnp.concatenate(
        [featureVector, pointCloudPose, PointCloudNormal, SH, zc], -1
    ).reshape(B * N, 128)
    boff = (jnp.arange(B, dtype=jnp.int32) * N)
    fps_flat = (fps_idx + boff[:, None]).reshape(-1)

    cent_rows = _sc_gather_build(B * S, 128, 32, 128)(tab, fps_flat)
    new_pose = cent_rows[:, 32:35].reshape(B, S, 3)
    new_normal = cent_rows[:, 35:38].reshape(B, S, 3)
    new_sh = cent_rows[:, 38:47].reshape(B, S, 9)

    nn_flat = _knn_pallas(new_pose, pointCloudPose).reshape(-1)
    cl_rows = _sc_gather_build(B * S * KNN, 128, 32, 512)(tab, nn_flat)

    M = B * S * KNN
    BM = 4096
    G = M // BM
    row2 = lambda a: a.reshape(1, -1)

    y1, st1 = pl.pallas_call(
        _p1_body,
        grid=(G,),
        in_specs=[
            pl.BlockSpec((BM, 128), lambda i: (i, 0)),
            pl.BlockSpec((BM // KNN, 128), lambda i: (i, 0)),
            pl.BlockSpec((5, 64), lambda i: (0, 0)),
            pl.BlockSpec((1, 64), lambda i: (0, 0)),
            pl.BlockSpec((64, 64), lambda i: (0, 0)),
            pl.BlockSpec((1, 64), lambda i: (0, 0)),
            pl.BlockSpec((32, 64), lambda i: (0, 0)),
            pl.BlockSpec((64, 64), lambda i: (0, 0)),
            pl.BlockSpec((1, 64), lambda i: (0, 0)),
        ],
        out_specs=[
            pl.BlockSpec((BM, 64), lambda i: (i, 0)),
            pl.BlockSpec((1, 2, 64), lambda i: (i, 0, 0)),
        ],
        out_shape=[
            jax.ShapeDtypeStruct((M, 64), jnp.float32),
            jax.ShapeDtypeStruct((G, 2, 64), jnp.float32),
        ],
        compiler_params=pltpu.CompilerParams(
            dimension_semantics=("parallel",)),
    )(cl_rows, cent_rows, rri_W1, row2(rri_b1), rri_W2, row2(rri_b2),
      conv_W1[0:32], conv_W1[32:96], row2(conv_b1))

    s1 = jnp.sum(st1, axis=0)
    mean1 = s1[0] / M
    var1 = s1[1] / M - mean1 * mean1
    sc1 = bn_g1 / jnp.sqrt(var1 + 1e-5)
    sh1 = bn_b1 - mean1 * sc1

    def bn_mm_pass(y, sc, sh, W, b, cout):
        return pl.pallas_call(
            _p2_body,
            grid=(G,),
            in_specs=[
                pl.BlockSpec((BM, y.shape[1]), lambda i: (i, 0)),
                pl.BlockSpec((1, y.shape[1]), lambda i: (0, 0)),
                pl.BlockSpec((1, y.shape[1]), lambda i: (0, 0)),
                pl.BlockSpec((y.shape[1], cout), lambda i: (0, 0)),
                pl.BlockSpec((1, cout), lambda i: (0, 0)),
            ],
            out_specs=[
                pl.BlockSpec((BM, cout), lambda i: (i, 0)),
                pl.BlockSpec((1, 2, cout), lambda i: (i, 0, 0)),
            ],
            out_shape=[
                jax.ShapeDtypeStruct((M, cout), jnp.float32),
                jax.ShapeDtypeStruct((G, 2, cout), jnp.float32),
            ],
            compiler_params=pltpu.CompilerParams(
                dimension_semantics=("parallel",)),
        )(y, row2(sc), row2(sh), W, row2(b))

    y2, st2 = bn_mm_pass(y1, sc1, sh1, conv_W2, conv_b2, 128)
    s2 = jnp.sum(st2, axis=0)
    mean2 = s2[0] / M
    var2 = s2[1] / M - mean2 * mean2
    sc2 = bn_g2 / jnp.sqrt(var2 + 1e-5)
    sh2 = bn_b2 - mean2 * sc2

    new_feat = pl.pallas_call(
        _p3_body,
        grid=(G,),
        in_specs=[
            pl.BlockSpec((BM, 128), lambda i: (i, 0)),
            pl.BlockSpec((1, 128), lambda i: (0, 0)),
            pl.BlockSpec((1, 128), lambda i: (0, 0)),
        ],
        out_specs=pl.BlockSpec((BM // KNN, 128), lambda i: (i, 0)),
        out_shape=jax.ShapeDtypeStruct((B * S, 128), jnp.float32),
        compiler_params=pltpu.CompilerParams(
            dimension_semantics=("parallel",)),
    )(y2, row2(sc2), row2(sh2)).reshape(B, S, 128)
    return (new_pose, new_feat, new_normal, new_sh)
